# Initial kernel scaffold; baseline (speedup 1.0000x reference)
import functools

import jax
import jax.numpy as jnp
from jax import lax
from jax.experimental import pallas as pl
from jax.experimental.pallas import tpu as pltpu
from jax.experimental.pallas import tpu_sc as plsc

_NUM_CORES = 2
_NUM_SUBCORES = 16
_NUM_WORKERS = _NUM_CORES * _NUM_SUBCORES

_CHUNK = 128
_NBUF = 2


def _gather_kernel(idx_hbm, table_hbm, out_hbm, idx_v, bufs, sems, *,
                   chunks_per_worker):
    wid = lax.axis_index("s") * _NUM_CORES + lax.axis_index("c")
    pltpu.sync_copy(idx_hbm.at[pl.ds(wid * chunks_per_worker, chunks_per_worker)],
                    idx_v)
    for b in range(_NBUF):
        pltpu.async_copy(table_hbm.at[idx_v.at[b]], bufs[b], sems[b])

    out_base = wid * chunks_per_worker * _CHUNK

    def step(i, carry):
        for b in range(_NBUF):
            j = i * _NBUF + b
            pltpu.make_async_copy(table_hbm.at[idx_v.at[j]], bufs[b],
                                  sems[b]).wait()
            nj = j + _NBUF

            @pl.when(nj < chunks_per_worker)
            def _():
                pltpu.async_copy(table_hbm.at[idx_v.at[nj]], bufs[b], sems[b])

            pltpu.sync_copy(bufs[b],
                            out_hbm.at[pl.ds(out_base + j * _CHUNK, _CHUNK)])
        return carry

    lax.fori_loop(0, chunks_per_worker // _NBUF, step, 0)


def kernel(indices, weight):
    b0, s0 = indices.shape
    vocab, dim = weight.shape
    total = b0 * s0
    assert total % (_NUM_WORKERS * _CHUNK) == 0
    chunks_per_worker = total // (_NUM_WORKERS * _CHUNK)

    idx2d = indices.reshape(total // _CHUNK, _CHUNK).astype(jnp.int32)

    mesh = plsc.VectorSubcoreMesh(core_axis_name="c", subcore_axis_name="s",
                                  num_cores=_NUM_CORES,
                                  num_subcores=_NUM_SUBCORES)
    grid_kernel = pl.kernel(
        functools.partial(_gather_kernel,
                          chunks_per_worker=chunks_per_worker),
        out_type=jax.ShapeDtypeStruct((total, dim), jnp.float32),
        mesh=mesh,
        scratch_types=[
            pltpu.VMEM((chunks_per_worker, _CHUNK), jnp.int32),
            [pltpu.VMEM((_CHUNK, dim), jnp.float32) for _ in range(_NBUF)],
            [pltpu.SemaphoreType.DMA for _ in range(_NBUF)],
        ],
    )
    out = grid_kernel(idx2d, weight)
    return out.reshape(b0, s0, dim)


# speedup vs baseline: 3.3272x; 3.3272x over previous
import functools

import jax
import jax.numpy as jnp
from jax import lax
from jax.experimental import pallas as pl
from jax.experimental.pallas import tpu as pltpu
from jax.experimental.pallas import tpu_sc as plsc

_NUM_CORES = 2
_NUM_SUBCORES = 16
_NUM_WORKERS = _NUM_CORES * _NUM_SUBCORES

_CHUNK = 128
_NBUF = 2


def _gather_kernel(idx_hbm, table_hbm, out_hbm, idx_v, bufs, sems, *,
                   chunks_per_worker):
    wid = lax.axis_index("s") * _NUM_CORES + lax.axis_index("c")
    pltpu.sync_copy(idx_hbm.at[wid], idx_v)
    for b in range(_NBUF):
        pltpu.async_copy(table_hbm.at[idx_v.at[b]], bufs[b], sems[b])

    out_base = wid * chunks_per_worker * _CHUNK

    def step(i, carry):
        for b in range(_NBUF):
            j = i * _NBUF + b
            pltpu.make_async_copy(table_hbm.at[idx_v.at[j]], bufs[b],
                                  sems[b]).wait()
            pltpu.sync_copy(bufs[b],
                            out_hbm.at[pl.ds(out_base + j * _CHUNK, _CHUNK)])
            nj = j + _NBUF

            @pl.when(nj < chunks_per_worker)
            def _():
                pltpu.async_copy(table_hbm.at[idx_v.at[nj]], bufs[b], sems[b])
        return carry

    lax.fori_loop(0, chunks_per_worker // _NBUF, step, 0)


def kernel(indices, weight):
    b0, s0 = indices.shape
    vocab, dim = weight.shape
    total = b0 * s0
    assert total % (_NUM_WORKERS * _CHUNK) == 0
    chunks_per_worker = total // (_NUM_WORKERS * _CHUNK)

    idx3d = indices.reshape(_NUM_WORKERS, chunks_per_worker,
                            _CHUNK).astype(jnp.int32)

    mesh = plsc.VectorSubcoreMesh(core_axis_name="c", subcore_axis_name="s",
                                  num_cores=_NUM_CORES,
                                  num_subcores=_NUM_SUBCORES)
    grid_kernel = pl.kernel(
        functools.partial(_gather_kernel,
                          chunks_per_worker=chunks_per_worker),
        out_type=jax.ShapeDtypeStruct((total, dim), jnp.float32),
        mesh=mesh,
        scratch_types=[
            pltpu.VMEM((chunks_per_worker, _CHUNK), jnp.int32),
            [pltpu.VMEM((_CHUNK, dim), jnp.float32) for _ in range(_NBUF)],
            [pltpu.SemaphoreType.DMA for _ in range(_NBUF)],
        ],
    )
    out = grid_kernel(idx3d, weight)
    return out.reshape(b0, s0, dim)


# speedup vs baseline: 3.3366x; 1.0028x over previous
import functools

import jax
import jax.numpy as jnp
from jax import lax
from jax.experimental import pallas as pl
from jax.experimental.pallas import tpu as pltpu
from jax.experimental.pallas import tpu_sc as plsc

_NUM_CORES = 2
_NUM_SUBCORES = 16
_NUM_WORKERS = _NUM_CORES * _NUM_SUBCORES

_CHUNK = 128
_NSLOT = 5
_LEAD = 3


def _gather_kernel(idx_hbm, table_hbm, out_hbm, idx_v, bufs, gsems, ssems, *,
                   chunks_per_worker):
    nct = chunks_per_worker
    wid = lax.axis_index("s") * _NUM_CORES + lax.axis_index("c")
    pltpu.sync_copy(idx_hbm.at[wid], idx_v)
    out_base = wid * nct * _CHUNK

    def g_start(j, b):
        pltpu.async_copy(table_hbm.at[idx_v.at[j]], bufs[b], gsems[b])

    def g_wait(j, b):
        pltpu.make_async_copy(table_hbm.at[idx_v.at[j]], bufs[b],
                              gsems[b]).wait()

    def s_start(j, b):
        pltpu.async_copy(bufs[b],
                         out_hbm.at[pl.ds(out_base + j * _CHUNK, _CHUNK)],
                         ssems[b])

    def s_wait(b):
        pltpu.make_async_copy(bufs[b], out_hbm.at[pl.ds(out_base, _CHUNK)],
                              ssems[b]).wait()

    for p in range(_LEAD):
        g_start(p, p)

    for b in range(_NSLOT):
        j = b
        g_wait(j, b)
        s_start(j, b)
        jn = j + _LEAD
        bn = jn % _NSLOT
        if jn >= _NSLOT:
            s_wait(bn)
        g_start(jn, bn)

    def step(i, carry):
        for b in range(_NSLOT):
            j = i * _NSLOT + b
            g_wait(j, b)
            s_start(j, b)
            jn = j + _LEAD
            bn = (b + _LEAD) % _NSLOT

            @pl.when(jn < nct)
            def _():
                s_wait(bn)
                g_start(jn, bn)

        return carry

    lax.fori_loop(1, nct // _NSLOT, step, 0)

    for b in range(_NSLOT):
        s_wait(b)


def kernel(indices, weight):
    b0, s0 = indices.shape
    vocab, dim = weight.shape
    total = b0 * s0
    assert total % (_NUM_WORKERS * _CHUNK) == 0
    chunks_per_worker = total // (_NUM_WORKERS * _CHUNK)
    assert chunks_per_worker % _NSLOT == 0

    idx3d = indices.reshape(_NUM_WORKERS, chunks_per_worker,
                            _CHUNK).astype(jnp.int32)

    mesh = plsc.VectorSubcoreMesh(core_axis_name="c", subcore_axis_name="s",
                                  num_cores=_NUM_CORES,
                                  num_subcores=_NUM_SUBCORES)
    grid_kernel = pl.kernel(
        functools.partial(_gather_kernel,
                          chunks_per_worker=chunks_per_worker),
        out_type=jax.ShapeDtypeStruct((total, dim), jnp.float32),
        mesh=mesh,
        scratch_types=[
            pltpu.VMEM((chunks_per_worker, _CHUNK), jnp.int32),
            [pltpu.VMEM((_CHUNK, dim), jnp.float32) for _ in range(_NSLOT)],
            [pltpu.SemaphoreType.DMA for _ in range(_NSLOT)],
            [pltpu.SemaphoreType.DMA for _ in range(_NSLOT)],
        ],
    )
    out = grid_kernel(idx3d, weight)
    return out.reshape(b0, s0, dim)


# speedup vs baseline: 5.9468x; 1.7823x over previous
import functools

import jax
import jax.numpy as jnp
from jax import lax
from jax.experimental import pallas as pl
from jax.experimental.pallas import tpu as pltpu
from jax.experimental.pallas import tpu_sc as plsc

_NUM_CORES = 2
_NUM_SUBCORES = 16
_NUM_WORKERS = _NUM_CORES * _NUM_SUBCORES

_NSLOT = 8
_LEAD = 4


def _gather_kernel(idx_hbm, table_hbm, out_hbm, idx_v, bufs, gsems, ssems, *,
                   rows_per_worker, seq):
    nrw = rows_per_worker
    wid = lax.axis_index("s") * _NUM_CORES + lax.axis_index("c")
    row0 = wid * nrw
    pltpu.sync_copy(idx_hbm.at[pl.ds(row0, nrw)], idx_v)

    def g_start(j, b):
        pltpu.async_copy(table_hbm.at[idx_v.at[j]], bufs[b], gsems[b])

    def g_wait(j, b):
        pltpu.make_async_copy(table_hbm.at[idx_v.at[j]], bufs[b],
                              gsems[b]).wait()

    def s_start(j, b):
        pltpu.async_copy(bufs[b], out_hbm.at[row0 + j], ssems[b])

    def s_wait(b):
        pltpu.make_async_copy(bufs[b], out_hbm.at[row0], ssems[b]).wait()

    for p in range(_LEAD):
        g_start(p, p)

    for j in range(_NSLOT):
        g_wait(j, j)
        s_start(j, j)
        jn = j + _LEAD
        bn = jn % _NSLOT
        if jn >= _NSLOT:
            s_wait(bn)
        g_start(jn, bn)

    def step(i, carry):
        for b in range(_NSLOT):
            j = i * _NSLOT + b
            g_wait(j, b)
            s_start(j, b)
            jn = j + _LEAD
            bn = (b + _LEAD) % _NSLOT

            @pl.when(jn < nrw)
            def _():
                s_wait(bn)
                g_start(jn, bn)

        return carry

    lax.fori_loop(1, nrw // _NSLOT, step, 0)

    for b in range(_NSLOT):
        s_wait(b)


def kernel(indices, weight):
    b0, seq = indices.shape
    vocab, dim = weight.shape
    assert b0 % _NUM_WORKERS == 0
    rows_per_worker = b0 // _NUM_WORKERS
    assert rows_per_worker % _NSLOT == 0

    idx = indices.astype(jnp.int32)

    mesh = plsc.VectorSubcoreMesh(core_axis_name="c", subcore_axis_name="s",
                                  num_cores=_NUM_CORES,
                                  num_subcores=_NUM_SUBCORES)
    grid_kernel = pl.kernel(
        functools.partial(_gather_kernel, rows_per_worker=rows_per_worker,
                          seq=seq),
        out_type=jax.ShapeDtypeStruct((b0, seq, dim), jnp.float32),
        mesh=mesh,
        scratch_types=[
            pltpu.VMEM((rows_per_worker, seq), jnp.int32),
            [pltpu.VMEM((seq, dim), jnp.float32) for _ in range(_NSLOT)],
            [pltpu.SemaphoreType.DMA for _ in range(_NSLOT)],
            [pltpu.SemaphoreType.DMA for _ in range(_NSLOT)],
        ],
    )
    return grid_kernel(idx, weight)


# speedup vs baseline: 5.9471x; 1.0001x over previous
import functools

import jax
import jax.numpy as jnp
from jax import lax
from jax.experimental import pallas as pl
from jax.experimental.pallas import tpu as pltpu
from jax.experimental.pallas import tpu_sc as plsc

_NUM_CORES = 2
_NUM_SUBCORES = 16
_NUM_WORKERS = _NUM_CORES * _NUM_SUBCORES

_NSLOT = 8
_LEAD = 4


def _gather_kernel(idx_hbm, table_hbm, out_hbm, idx_v, bufs, gsems, ssems, *,
                   rows_per_worker, seq):
    nrw = rows_per_worker
    wid = lax.axis_index("s") * _NUM_CORES + lax.axis_index("c")
    row0 = wid * nrw
    pltpu.sync_copy(idx_hbm.at[pl.ds(row0, nrw)], idx_v)

    def g_start(j, b):
        pltpu.async_copy(table_hbm.at[idx_v.at[j]], bufs[b], gsems[b])

    def g_wait(j, b):
        pltpu.make_async_copy(table_hbm.at[idx_v.at[j]], bufs[b],
                              gsems[b]).wait()

    def s_start(j, b):
        pltpu.async_copy(bufs[b], out_hbm.at[row0 + j], ssems[b])

    def s_wait(b):
        pltpu.make_async_copy(bufs[b], out_hbm.at[row0], ssems[b]).wait()

    for p in range(_LEAD):
        g_start(p, p)

    for j in range(_NSLOT):
        g_wait(j, j)
        s_start(j, j)
        jn = j + _LEAD
        bn = jn % _NSLOT
        if jn >= _NSLOT:
            s_wait(bn)
        g_start(jn, bn)

    def step(i, carry):
        for b in range(_NSLOT):
            j = i * _NSLOT + b
            g_wait(j, b)
            s_start(j, b)
            jn = j + _LEAD
            bn = (b + _LEAD) % _NSLOT

            @pl.when(jn < nrw)
            def _():
                s_wait(bn)
                g_start(jn, bn)

        return carry

    lax.fori_loop(1, nrw // _NSLOT, step, 0)

    for b in range(_NSLOT):
        s_wait(b)


def kernel(indices, weight):
    b0, seq = indices.shape
    vocab, dim = weight.shape
    assert b0 % _NUM_WORKERS == 0
    rows_per_worker = b0 // _NUM_WORKERS
    assert rows_per_worker % _NSLOT == 0

    idx = indices.astype(jnp.int32)

    mesh = plsc.VectorSubcoreMesh(core_axis_name="c", subcore_axis_name="s",
                                  num_cores=_NUM_CORES,
                                  num_subcores=_NUM_SUBCORES)
    grid_kernel = pl.kernel(
        functools.partial(_gather_kernel, rows_per_worker=rows_per_worker,
                          seq=seq),
        out_type=jax.ShapeDtypeStruct((b0, seq, dim), jnp.float32),
        mesh=mesh,
        compiler_params=pltpu.CompilerParams(use_tc_tiling_on_sc=True),
        scratch_types=[
            pltpu.VMEM((rows_per_worker, seq), jnp.int32),
            [pltpu.VMEM((seq, dim), jnp.float32) for _ in range(_NSLOT)],
            [pltpu.SemaphoreType.DMA for _ in range(_NSLOT)],
            [pltpu.SemaphoreType.DMA for _ in range(_NSLOT)],
        ],
    )
    return grid_kernel(idx, weight)


# speedup vs baseline: 10.7255x; 1.8035x over previous
import functools

import jax
import jax.numpy as jnp
from jax import lax
from jax.experimental import pallas as pl
from jax.experimental.pallas import tpu as pltpu
from jax.experimental.pallas import tpu_sc as plsc

_NUM_CORES = 2
_NUM_SUBCORES = 16
_NUM_WORKERS = _NUM_CORES * _NUM_SUBCORES

_NSLOT = 5
_LEAD = 3


def _gather_kernel(idx_hbm, table_hbm, out_hbm, idx_v, bufs, gsems, ssems, *,
                   seq, bpw):
    wid = lax.axis_index("s") * _NUM_CORES + lax.axis_index("c")
    col0 = wid * bpw
    pltpu.sync_copy(idx_hbm.at[:, pl.ds(col0, bpw)], idx_v)

    def g_start(j, b):
        pltpu.async_copy(table_hbm.at[idx_v.at[j]], bufs[b], gsems[b])

    def g_wait(j, b):
        pltpu.make_async_copy(table_hbm.at[idx_v.at[j]], bufs[b],
                              gsems[b]).wait()

    def s_start(j, b):
        pltpu.async_copy(bufs[b], out_hbm.at[j, pl.ds(col0, bpw)], ssems[b])

    def s_wait(b):
        pltpu.make_async_copy(bufs[b], out_hbm.at[0, pl.ds(col0, bpw)],
                              ssems[b]).wait()

    for p in range(_LEAD):
        g_start(p, p)

    for j in range(_NSLOT):
        g_wait(j, j)
        s_start(j, j)
        jn = j + _LEAD
        bn = jn % _NSLOT
        if jn >= _NSLOT:
            s_wait(bn)
        g_start(jn, bn)

    def step(i, carry):
        for b in range(_NSLOT):
            j = i * _NSLOT + b
            g_wait(j, b)
            s_start(j, b)
            jn = j + _LEAD
            bn = (b + _LEAD) % _NSLOT

            @pl.when(jn < seq)
            def _():
                s_wait(bn)
                g_start(jn, bn)

        return carry

    lax.fori_loop(1, seq // _NSLOT, step, 0)

    for b in range(_NSLOT):
        s_wait(b)


def kernel(indices, weight):
    b0, seq = indices.shape
    vocab, dim = weight.shape
    assert b0 % _NUM_WORKERS == 0
    bpw = b0 // _NUM_WORKERS
    assert seq % _NSLOT == 0

    idx_t = indices.astype(jnp.int32).T

    mesh = plsc.VectorSubcoreMesh(core_axis_name="c", subcore_axis_name="s",
                                  num_cores=_NUM_CORES,
                                  num_subcores=_NUM_SUBCORES)
    grid_kernel = pl.kernel(
        functools.partial(_gather_kernel, seq=seq, bpw=bpw),
        out_type=jax.ShapeDtypeStruct((seq, b0, dim), jnp.float32),
        mesh=mesh,
        scratch_types=[
            pltpu.VMEM((seq, bpw), jnp.int32),
            [pltpu.VMEM((bpw, dim), jnp.float32) for _ in range(_NSLOT)],
            [pltpu.SemaphoreType.DMA for _ in range(_NSLOT)],
            [pltpu.SemaphoreType.DMA for _ in range(_NSLOT)],
        ],
    )
    out = grid_kernel(idx_t, weight)
    return out.transpose(1, 0, 2)
